# baseline (speedup 1.0000x reference)
import functools

import jax
import jax.numpy as jnp
from jax.experimental import pallas as pl
from jax.experimental.pallas import tpu as pltpu

_P = 256
_D = 768
_V = 49408
_VB = 3088
_NB = 8


def _block_min(c, a, base):
    dot = jax.lax.dot_general(
        c, a, (((1,), (1,)), ((), ())),
        preferred_element_type=jnp.float32)
    b2 = jnp.sum(c * c, axis=1, keepdims=True)
    s = b2 - 2.0 * dot
    m = jnp.min(s, axis=0, keepdims=True)
    iota = jax.lax.broadcasted_iota(jnp.int32, (_VB, _P), 0)
    li = jnp.min(jnp.where(s == m, iota, _V), axis=0, keepdims=True)
    return m, li + base


def _merge(acc_val, acc_idx, m, gi):
    av = acc_val[...]
    ai = acc_idx[...]
    better = (m < av) | ((m == av) & (gi < ai))
    acc_val[...] = jnp.where(better, m, av)
    acc_idx[...] = jnp.where(better, gi, ai)


def _nn_kernel(a_ref, c1_ref, c2_ref, idx_ref, acc_val, acc_idx):
    j = pl.program_id(0)

    @pl.when(j == 0)
    def _init():
        acc_val[...] = jnp.full((1, _P), jnp.inf, dtype=jnp.float32)
        acc_idx[...] = jnp.zeros((1, _P), dtype=jnp.int32)

    a = a_ref[...]
    m1, g1 = _block_min(c1_ref[...], a, j * _VB)
    _merge(acc_val, acc_idx, m1, g1)
    m2, g2 = _block_min(c2_ref[...], a, (j + _NB) * _VB)
    _merge(acc_val, acc_idx, m2, g2)

    @pl.when(j == _NB - 1)
    def _done():
        idx_ref[...] = acc_idx[...]


def _nearest_ids(prompt_embs, clip_embs):
    ids2d = pl.pallas_call(
        _nn_kernel,
        grid=(_NB,),
        in_specs=[
            pl.BlockSpec((_P, _D), lambda j: (0, 0)),
            pl.BlockSpec((_VB, _D), lambda j: (j, 0)),
            pl.BlockSpec((_VB, _D), lambda j: (j + _NB, 0)),
        ],
        out_specs=pl.BlockSpec((1, _P), lambda j: (0, 0)),
        out_shape=jax.ShapeDtypeStruct((1, _P), jnp.int32),
        scratch_shapes=[
            pltpu.VMEM((1, _P), jnp.float32),
            pltpu.VMEM((1, _P), jnp.int32),
        ],
        compiler_params=pltpu.CompilerParams(
            dimension_semantics=("arbitrary",),
        ),
    )(prompt_embs, clip_embs, clip_embs)
    return ids2d.reshape(_P)


@jax.jit
def kernel(prompt_embs, clip_embs):
    ids = _nearest_ids(prompt_embs, clip_embs)
    return (prompt_embs, prompt_embs, ids)


# speedup vs baseline: 1.0804x; 1.0804x over previous
import functools

import jax
import jax.numpy as jnp
from jax.experimental import pallas as pl
from jax.experimental.pallas import tpu as pltpu

_P = 256
_D = 768
_V = 49408
_VB = 6176


def _nn_kernel(a_ref, c_ref, idx_ref, acc_val, acc_idx, *, n_blocks):
    j = pl.program_id(0)

    @pl.when(j == 0)
    def _init():
        acc_val[...] = jnp.full((1, _P), jnp.inf, dtype=jnp.float32)
        acc_idx[...] = jnp.zeros((1, _P), dtype=jnp.int32)

    c = c_ref[...]
    a = a_ref[...]
    dot = jax.lax.dot_general(
        c, a, (((1,), (1,)), ((), ())),
        preferred_element_type=jnp.float32)
    b2 = jnp.sum(c * c, axis=1, keepdims=True)
    s = b2 - 2.0 * dot

    m = jnp.min(s, axis=0, keepdims=True)
    iota = jax.lax.broadcasted_iota(jnp.int32, (_VB, _P), 0)
    li = jnp.min(jnp.where(s == m, iota, _V), axis=0, keepdims=True)
    gi = li + j * _VB

    better = m < acc_val[...]
    acc_val[...] = jnp.where(better, m, acc_val[...])
    acc_idx[...] = jnp.where(better, gi, acc_idx[...])

    @pl.when(j == n_blocks - 1)
    def _done():
        idx_ref[...] = acc_idx[...]


def _nearest_ids(prompt_embs, clip_embs):
    n_blocks = _V // _VB
    ids2d = pl.pallas_call(
        functools.partial(_nn_kernel, n_blocks=n_blocks),
        grid=(n_blocks,),
        in_specs=[
            pl.BlockSpec((_P, _D), lambda j: (0, 0)),
            pl.BlockSpec((_VB, _D), lambda j: (j, 0)),
        ],
        out_specs=pl.BlockSpec((1, _P), lambda j: (0, 0)),
        out_shape=jax.ShapeDtypeStruct((1, _P), jnp.int32),
        scratch_shapes=[
            pltpu.VMEM((1, _P), jnp.float32),
            pltpu.VMEM((1, _P), jnp.int32),
        ],
        compiler_params=pltpu.CompilerParams(
            dimension_semantics=("arbitrary",),
        ),
    )(prompt_embs, clip_embs)
    return ids2d.reshape(_P)


@jax.jit
def kernel(prompt_embs, clip_embs):
    ids = _nearest_ids(prompt_embs, clip_embs)
    return (prompt_embs, prompt_embs, ids)
